# initial kernel scaffold (unmeasured)
import jax
import jax.numpy as jnp
from jax import lax
from jax.experimental import pallas as pl
from jax.experimental.pallas import tpu as pltpu


def kernel(
    x,
):
    def body(*refs):
        pass

    out_shape = jax.ShapeDtypeStruct(..., jnp.float32)
    return pl.pallas_call(body, out_shape=out_shape)(...)


# baseline (device time: 595897 ns/iter reference)
import jax
import jax.numpy as jnp
from jax import lax
from jax.experimental import pallas as pl
from jax.experimental.pallas import tpu as pltpu

N_Z = 4


def kernel(x):
    m_per, n = x.shape
    half = m_per // 2

    def body(x_ref, out_ref, copy_sem, send_sems, recv_sems):
        my_x = lax.axis_index("x")
        my_y = lax.axis_index("y")
        my_z = lax.axis_index("z")
        left = (my_z + N_Z - 1) % N_Z
        right = (my_z + 1) % N_Z

        barrier_sem = pltpu.get_barrier_semaphore()
        for nbr in (left, right):
            pl.semaphore_signal(
                barrier_sem,
                inc=1,
                device_id=(my_x, my_y, nbr),
                device_id_type=pl.DeviceIdType.MESH,
            )
        pl.semaphore_wait(barrier_sem, 2)

        d_r1 = pltpu.make_async_remote_copy(
            src_ref=x_ref,
            dst_ref=out_ref.at[pl.ds(my_z * m_per, m_per), :],
            send_sem=send_sems.at[0],
            recv_sem=recv_sems.at[0],
            device_id=(my_x, my_y, right),
            device_id_type=pl.DeviceIdType.MESH,
        )
        d_l1 = pltpu.make_async_remote_copy(
            src_ref=x_ref,
            dst_ref=out_ref.at[pl.ds(my_z * m_per, m_per), :],
            send_sem=send_sems.at[1],
            recv_sem=recv_sems.at[1],
            device_id=(my_x, my_y, left),
            device_id_type=pl.DeviceIdType.MESH,
        )
        d_r1.start()
        d_l1.start()

        cp = pltpu.make_async_copy(
            x_ref, out_ref.at[pl.ds(my_z * m_per, m_per), :], copy_sem
        )
        cp.start()

        d_r1.wait_recv()
        d_r2 = pltpu.make_async_remote_copy(
            src_ref=out_ref.at[pl.ds(left * m_per, half), :],
            dst_ref=out_ref.at[pl.ds(left * m_per, half), :],
            send_sem=send_sems.at[2],
            recv_sem=recv_sems.at[2],
            device_id=(my_x, my_y, right),
            device_id_type=pl.DeviceIdType.MESH,
        )
        d_r2.start()

        d_l1.wait_recv()
        d_l2 = pltpu.make_async_remote_copy(
            src_ref=out_ref.at[pl.ds(right * m_per + half, half), :],
            dst_ref=out_ref.at[pl.ds(right * m_per + half, half), :],
            send_sem=send_sems.at[3],
            recv_sem=recv_sems.at[3],
            device_id=(my_x, my_y, left),
            device_id_type=pl.DeviceIdType.MESH,
        )
        d_l2.start()

        cp.wait()
        d_r2.wait_recv()
        d_l2.wait_recv()
        d_r1.wait_send()
        d_l1.wait_send()
        d_r2.wait_send()
        d_l2.wait_send()

    return pl.pallas_call(
        body,
        out_shape=jax.ShapeDtypeStruct((N_Z * m_per, n), x.dtype),
        in_specs=[pl.BlockSpec(memory_space=pl.ANY)],
        out_specs=pl.BlockSpec(memory_space=pl.ANY),
        scratch_shapes=[
            pltpu.SemaphoreType.DMA,
            pltpu.SemaphoreType.DMA((4,)),
            pltpu.SemaphoreType.DMA((4,)),
        ],
        compiler_params=pltpu.CompilerParams(collective_id=0),
    )(x)


# device time: 422035 ns/iter; 1.4120x vs baseline; 1.4120x over previous
import jax
import jax.numpy as jnp
from jax import lax
from jax.experimental import pallas as pl
from jax.experimental.pallas import tpu as pltpu

N_Z = 4
Q = 1


def kernel(x):
    m_per, n = x.shape
    half = m_per // 2
    seg = half // Q

    def body(
        x_ref,
        out_ref,
        copy_sem,
        zr_send,
        zr_recv,
        zl_send,
        zl_recv,
        xs_send,
        xs_recv,
    ):
        my_x = lax.axis_index("x")
        my_y = lax.axis_index("y")
        my_z = lax.axis_index("z")
        partner = 1 - my_x
        h_off = my_x * half

        zright = (my_z + 1) % N_Z
        zleft = (my_z + N_Z - 1) % N_Z

        can_send_right = my_z <= N_Z - 2
        can_send_left = my_z >= 1

        barrier_sem = pltpu.get_barrier_semaphore()
        for dev in ((partner, my_y, my_z), (my_x, my_y, zright), (my_x, my_y, zleft)):
            pl.semaphore_signal(
                barrier_sem,
                inc=1,
                device_id=dev,
                device_id_type=pl.DeviceIdType.MESH,
            )
        pl.semaphore_wait(barrier_sem, 3)

        cp = pltpu.make_async_copy(
            x_ref, out_ref.at[pl.ds(my_z * m_per, m_per), :], copy_sem
        )
        cp.start()

        def z_desc(step, g, origin, to_right):
            row = origin * m_per + h_off + g * seg
            if step == 0:
                src = x_ref.at[pl.ds(h_off + g * seg, seg), :]
            else:
                src = out_ref.at[pl.ds(row, seg), :]
            return pltpu.make_async_remote_copy(
                src_ref=src,
                dst_ref=out_ref.at[pl.ds(row, seg), :],
                send_sem=(zr_send if to_right else zl_send).at[step, g],
                recv_sem=(zr_recv if to_right else zl_recv).at[step, g],
                device_id=(my_x, my_y, zright if to_right else zleft),
                device_id_type=pl.DeviceIdType.MESH,
            )

        def x_desc(k, g, origin):
            row = origin * m_per + h_off + g * seg
            return pltpu.make_async_remote_copy(
                src_ref=out_ref.at[pl.ds(row, seg), :],
                dst_ref=out_ref.at[pl.ds(row, seg), :],
                send_sem=xs_send.at[k, g],
                recv_sem=xs_recv.at[k, g],
                device_id=(partner, my_y, my_z),
                device_id_type=pl.DeviceIdType.MESH,
            )

        for g in range(Q):

            @pl.when(can_send_right)
            def _(g=g):
                z_desc(0, g, my_z, True).start()

            @pl.when(can_send_left)
            def _(g=g):
                z_desc(0, g, my_z, False).start()

        for s in range(N_Z - 1):
            recv_r = my_z >= s + 1
            recv_l = my_z + s + 1 <= N_Z - 1
            o_r = my_z - 1 - s
            o_l = my_z + 1 + s
            for g in range(Q):

                @pl.when(recv_r)
                def _(s=s, g=g, o_r=o_r):
                    z_desc(s, g, o_r, True).wait_recv()
                    x_desc(s, g, o_r).start()
                    if s + 1 <= N_Z - 2:

                        @pl.when(can_send_right)
                        def _():
                            z_desc(s + 1, g, o_r, True).start()

                @pl.when(recv_l)
                def _(s=s, g=g, o_l=o_l):
                    z_desc(s, g, o_l, False).wait_recv()
                    x_desc(3 + s, g, o_l).start()
                    if s + 1 <= N_Z - 2:

                        @pl.when(can_send_left)
                        def _():
                            z_desc(s + 1, g, o_l, False).start()

        for s in range(N_Z - 1):
            recv_r = my_z >= s + 1
            recv_l = my_z + s + 1 <= N_Z - 1
            for g in range(Q):

                @pl.when(recv_r)
                def _(s=s, g=g):
                    x_desc(s, g, 0).wait_recv()

                @pl.when(recv_l)
                def _(s=s, g=g):
                    x_desc(3 + s, g, 0).wait_recv()

        cp.wait()

        for s in range(N_Z - 1):
            send_r = can_send_right & (my_z >= s)
            send_l = can_send_left & (my_z + s <= N_Z - 1)
            recv_r = my_z >= s + 1
            recv_l = my_z + s + 1 <= N_Z - 1
            for g in range(Q):

                @pl.when(send_r)
                def _(s=s, g=g):
                    z_desc(s, g, 0, True).wait_send()

                @pl.when(send_l)
                def _(s=s, g=g):
                    z_desc(s, g, 0, False).wait_send()

                @pl.when(recv_r)
                def _(s=s, g=g):
                    x_desc(s, g, 0).wait_send()

                @pl.when(recv_l)
                def _(s=s, g=g):
                    x_desc(3 + s, g, 0).wait_send()

    return pl.pallas_call(
        body,
        out_shape=jax.ShapeDtypeStruct((N_Z * m_per, n), x.dtype),
        in_specs=[pl.BlockSpec(memory_space=pl.ANY)],
        out_specs=pl.BlockSpec(memory_space=pl.ANY),
        scratch_shapes=[
            pltpu.SemaphoreType.DMA,
            pltpu.SemaphoreType.DMA((N_Z - 1, Q)),
            pltpu.SemaphoreType.DMA((N_Z - 1, Q)),
            pltpu.SemaphoreType.DMA((N_Z - 1, Q)),
            pltpu.SemaphoreType.DMA((N_Z - 1, Q)),
            pltpu.SemaphoreType.DMA((2 * (N_Z - 1), Q)),
            pltpu.SemaphoreType.DMA((2 * (N_Z - 1), Q)),
        ],
        compiler_params=pltpu.CompilerParams(collective_id=0),
    )(x)


# device time: 351377 ns/iter; 1.6959x vs baseline; 1.2011x over previous
import jax
import jax.numpy as jnp
from jax import lax
from jax.experimental import pallas as pl
from jax.experimental.pallas import tpu as pltpu

N_Z = 4
Q = 4


def kernel(x):
    m_per, n = x.shape
    half = m_per // 2
    seg = half // Q

    def body(
        x_ref,
        out_ref,
        copy_sem,
        zr_send,
        zr_recv,
        zl_send,
        zl_recv,
        xs_send,
        xs_recv,
    ):
        my_x = lax.axis_index("x")
        my_y = lax.axis_index("y")
        my_z = lax.axis_index("z")
        partner = 1 - my_x
        h_off = my_x * half

        zright = (my_z + 1) % N_Z
        zleft = (my_z + N_Z - 1) % N_Z

        can_send_right = my_z <= N_Z - 2
        can_send_left = my_z >= 1

        barrier_sem = pltpu.get_barrier_semaphore()
        for dev in ((partner, my_y, my_z), (my_x, my_y, zright), (my_x, my_y, zleft)):
            pl.semaphore_signal(
                barrier_sem,
                inc=1,
                device_id=dev,
                device_id_type=pl.DeviceIdType.MESH,
            )
        pl.semaphore_wait(barrier_sem, 3)

        cp = pltpu.make_async_copy(
            x_ref, out_ref.at[pl.ds(my_z * m_per, m_per), :], copy_sem
        )
        cp.start()

        def z_desc(step, g, origin, to_right):
            row = origin * m_per + h_off + g * seg
            if step == 0:
                src = x_ref.at[pl.ds(h_off + g * seg, seg), :]
            else:
                src = out_ref.at[pl.ds(row, seg), :]
            return pltpu.make_async_remote_copy(
                src_ref=src,
                dst_ref=out_ref.at[pl.ds(row, seg), :],
                send_sem=(zr_send if to_right else zl_send).at[step, g],
                recv_sem=(zr_recv if to_right else zl_recv).at[step, g],
                device_id=(my_x, my_y, zright if to_right else zleft),
                device_id_type=pl.DeviceIdType.MESH,
            )

        def x_desc(k, g, origin):
            row = origin * m_per + h_off + g * seg
            return pltpu.make_async_remote_copy(
                src_ref=out_ref.at[pl.ds(row, seg), :],
                dst_ref=out_ref.at[pl.ds(row, seg), :],
                send_sem=xs_send.at[k, g],
                recv_sem=xs_recv.at[k, g],
                device_id=(partner, my_y, my_z),
                device_id_type=pl.DeviceIdType.MESH,
            )

        for g in range(Q):

            @pl.when(can_send_right)
            def _(g=g):
                z_desc(0, g, my_z, True).start()

            @pl.when(can_send_left)
            def _(g=g):
                z_desc(0, g, my_z, False).start()

        for s in range(N_Z - 1):
            recv_r = my_z >= s + 1
            recv_l = my_z + s + 1 <= N_Z - 1
            o_r = my_z - 1 - s
            o_l = my_z + 1 + s
            for g in range(Q):

                @pl.when(recv_r)
                def _(s=s, g=g, o_r=o_r):
                    z_desc(s, g, o_r, True).wait_recv()
                    x_desc(s, g, o_r).start()
                    if s + 1 <= N_Z - 2:

                        @pl.when(can_send_right)
                        def _():
                            z_desc(s + 1, g, o_r, True).start()

                @pl.when(recv_l)
                def _(s=s, g=g, o_l=o_l):
                    z_desc(s, g, o_l, False).wait_recv()
                    x_desc(3 + s, g, o_l).start()
                    if s + 1 <= N_Z - 2:

                        @pl.when(can_send_left)
                        def _():
                            z_desc(s + 1, g, o_l, False).start()

        for s in range(N_Z - 1):
            recv_r = my_z >= s + 1
            recv_l = my_z + s + 1 <= N_Z - 1
            for g in range(Q):

                @pl.when(recv_r)
                def _(s=s, g=g):
                    x_desc(s, g, 0).wait_recv()

                @pl.when(recv_l)
                def _(s=s, g=g):
                    x_desc(3 + s, g, 0).wait_recv()

        cp.wait()

        for s in range(N_Z - 1):
            send_r = can_send_right & (my_z >= s)
            send_l = can_send_left & (my_z + s <= N_Z - 1)
            recv_r = my_z >= s + 1
            recv_l = my_z + s + 1 <= N_Z - 1
            for g in range(Q):

                @pl.when(send_r)
                def _(s=s, g=g):
                    z_desc(s, g, 0, True).wait_send()

                @pl.when(send_l)
                def _(s=s, g=g):
                    z_desc(s, g, 0, False).wait_send()

                @pl.when(recv_r)
                def _(s=s, g=g):
                    x_desc(s, g, 0).wait_send()

                @pl.when(recv_l)
                def _(s=s, g=g):
                    x_desc(3 + s, g, 0).wait_send()

    return pl.pallas_call(
        body,
        out_shape=jax.ShapeDtypeStruct((N_Z * m_per, n), x.dtype),
        in_specs=[pl.BlockSpec(memory_space=pl.ANY)],
        out_specs=pl.BlockSpec(memory_space=pl.ANY),
        scratch_shapes=[
            pltpu.SemaphoreType.DMA,
            pltpu.SemaphoreType.DMA((N_Z - 1, Q)),
            pltpu.SemaphoreType.DMA((N_Z - 1, Q)),
            pltpu.SemaphoreType.DMA((N_Z - 1, Q)),
            pltpu.SemaphoreType.DMA((N_Z - 1, Q)),
            pltpu.SemaphoreType.DMA((2 * (N_Z - 1), Q)),
            pltpu.SemaphoreType.DMA((2 * (N_Z - 1), Q)),
        ],
        compiler_params=pltpu.CompilerParams(collective_id=0),
    )(x)


# device time: 273882 ns/iter; 2.1757x vs baseline; 1.2830x over previous
import jax
import jax.numpy as jnp
from jax import lax
from jax.experimental import pallas as pl
from jax.experimental.pallas import tpu as pltpu

N_Z = 4
Q = 4
QH = Q // 2


def kernel(x):
    m_per, n = x.shape
    m4 = m_per // 4
    seg = m4 // Q

    def body(
        x_ref,
        out_ref,
        copy_sem,
        zr_send,
        zr_recv,
        zl_send,
        zl_recv,
        yd_send,
        yd_recv,
        xd_send,
        xd_recv,
        xr_send,
        xr_recv,
        yr_send,
        yr_recv,
    ):
        my_x = lax.axis_index("x")
        my_y = lax.axis_index("y")
        my_z = lax.axis_index("z")
        x_partner = 1 - my_x
        y_pair = my_y - (my_y % 2) * 2 + 1
        par = my_y % 2

        q_off = (2 * my_x + par) * m4
        qy_off = (2 * my_x + (1 - par)) * m4
        qx_off = (2 * x_partner + par) * m4
        qd_off = (2 * x_partner + (1 - par)) * m4

        zright = (my_z + 1) % N_Z
        zleft = (my_z + N_Z - 1) % N_Z

        can_send_right = my_z <= N_Z - 2
        can_send_left = my_z >= 1

        barrier_sem = pltpu.get_barrier_semaphore()
        for dev in (
            (x_partner, my_y, my_z),
            (my_x, y_pair, my_z),
            (my_x, my_y, zright),
            (my_x, my_y, zleft),
        ):
            pl.semaphore_signal(
                barrier_sem,
                inc=1,
                device_id=dev,
                device_id_type=pl.DeviceIdType.MESH,
            )
        pl.semaphore_wait(barrier_sem, 4)

        cp = pltpu.make_async_copy(
            x_ref, out_ref.at[pl.ds(my_z * m_per, m_per), :], copy_sem
        )
        cp.start()

        def z_desc(step, g, origin, to_right):
            row = origin * m_per + q_off + g * seg
            if step == 0:
                src = x_ref.at[pl.ds(q_off + g * seg, seg), :]
            else:
                src = out_ref.at[pl.ds(row, seg), :]
            return pltpu.make_async_remote_copy(
                src_ref=src,
                dst_ref=out_ref.at[pl.ds(row, seg), :],
                send_sem=(zr_send if to_right else zl_send).at[step, g],
                recv_sem=(zr_recv if to_right else zl_recv).at[step, g],
                device_id=(my_x, my_y, zright if to_right else zleft),
                device_id_type=pl.DeviceIdType.MESH,
            )

        def lat_desc(send_sems, recv_sems, k, g, row_off, to_x, row_g=None):
            row = row_off + (g if row_g is None else row_g) * seg
            dev = (x_partner, my_y, my_z) if to_x else (my_x, y_pair, my_z)
            return pltpu.make_async_remote_copy(
                src_ref=out_ref.at[pl.ds(row, seg), :],
                dst_ref=out_ref.at[pl.ds(row, seg), :],
                send_sem=send_sems.at[k, g],
                recv_sem=recv_sems.at[k, g],
                device_id=dev,
                device_id_type=pl.DeviceIdType.MESH,
            )

        for g in range(Q):

            @pl.when(can_send_right)
            def _(g=g):
                z_desc(0, g, my_z, True).start()

            @pl.when(can_send_left)
            def _(g=g):
                z_desc(0, g, my_z, False).start()

        for s in range(N_Z - 1):
            recv_r = my_z >= s + 1
            recv_l = my_z + s + 1 <= N_Z - 1
            o_r = my_z - 1 - s
            o_l = my_z + 1 + s
            for g in range(Q):

                @pl.when(recv_r)
                def _(s=s, g=g, o_r=o_r):
                    z_desc(s, g, o_r, True).wait_recv()
                    lat_desc(yd_send, yd_recv, s, g, o_r * m_per + q_off, False).start()
                    lat_desc(xd_send, xd_recv, s, g, o_r * m_per + q_off, True).start()
                    if s + 1 <= N_Z - 2:

                        @pl.when(can_send_right)
                        def _():
                            z_desc(s + 1, g, o_r, True).start()

                @pl.when(recv_l)
                def _(s=s, g=g, o_l=o_l):
                    z_desc(s, g, o_l, False).wait_recv()
                    lat_desc(yd_send, yd_recv, 3 + s, g, o_l * m_per + q_off, False).start()
                    lat_desc(xd_send, xd_recv, 3 + s, g, o_l * m_per + q_off, True).start()
                    if s + 1 <= N_Z - 2:

                        @pl.when(can_send_left)
                        def _():
                            z_desc(s + 1, g, o_l, False).start()

        for s in range(N_Z - 1):
            recv_r = my_z >= s + 1
            recv_l = my_z + s + 1 <= N_Z - 1
            o_r = my_z - 1 - s
            o_l = my_z + 1 + s
            for g in range(Q):

                @pl.when(recv_r)
                def _(s=s, g=g, o_r=o_r):
                    lat_desc(yd_send, yd_recv, s, g, o_r * m_per + qy_off, False).wait_recv()
                    if g < QH:
                        lat_desc(xr_send, xr_recv, s, g, o_r * m_per + qy_off, True).start()

                @pl.when(recv_l)
                def _(s=s, g=g, o_l=o_l):
                    lat_desc(yd_send, yd_recv, 3 + s, g, o_l * m_per + qy_off, False).wait_recv()
                    if g < QH:
                        lat_desc(xr_send, xr_recv, 3 + s, g, o_l * m_per + qy_off, True).start()

                @pl.when(recv_r)
                def _(s=s, g=g, o_r=o_r):
                    lat_desc(xd_send, xd_recv, s, g, o_r * m_per + qx_off, True).wait_recv()
                    if g >= QH:
                        lat_desc(yr_send, yr_recv, s, g - QH, o_r * m_per + qx_off, False, row_g=g).start()

                @pl.when(recv_l)
                def _(s=s, g=g, o_l=o_l):
                    lat_desc(xd_send, xd_recv, 3 + s, g, o_l * m_per + qx_off, True).wait_recv()
                    if g >= QH:
                        lat_desc(yr_send, yr_recv, 3 + s, g - QH, o_l * m_per + qx_off, False, row_g=g).start()

        for s in range(N_Z - 1):
            recv_r = my_z >= s + 1
            recv_l = my_z + s + 1 <= N_Z - 1
            for g in range(QH):

                @pl.when(recv_r)
                def _(s=s, g=g):
                    lat_desc(xr_send, xr_recv, s, g, qd_off, True).wait_recv()
                    lat_desc(yr_send, yr_recv, s, g, qd_off, False).wait_recv()

                @pl.when(recv_l)
                def _(s=s, g=g):
                    lat_desc(xr_send, xr_recv, 3 + s, g, qd_off, True).wait_recv()
                    lat_desc(yr_send, yr_recv, 3 + s, g, qd_off, False).wait_recv()

        cp.wait()

        for s in range(N_Z - 1):
            send_r = can_send_right & (my_z >= s)
            send_l = can_send_left & (my_z + s <= N_Z - 1)
            recv_r = my_z >= s + 1
            recv_l = my_z + s + 1 <= N_Z - 1
            for g in range(Q):

                @pl.when(send_r)
                def _(s=s, g=g):
                    z_desc(s, g, 0, True).wait_send()

                @pl.when(send_l)
                def _(s=s, g=g):
                    z_desc(s, g, 0, False).wait_send()

                @pl.when(recv_r)
                def _(s=s, g=g):
                    lat_desc(yd_send, yd_recv, s, g, 0, False).wait_send()
                    lat_desc(xd_send, xd_recv, s, g, 0, True).wait_send()
                    if g < QH:
                        lat_desc(xr_send, xr_recv, s, g, 0, True).wait_send()
                        lat_desc(yr_send, yr_recv, s, g, 0, False).wait_send()

                @pl.when(recv_l)
                def _(s=s, g=g):
                    lat_desc(yd_send, yd_recv, 3 + s, g, 0, False).wait_send()
                    lat_desc(xd_send, xd_recv, 3 + s, g, 0, True).wait_send()
                    if g < QH:
                        lat_desc(xr_send, xr_recv, 3 + s, g, 0, True).wait_send()
                        lat_desc(yr_send, yr_recv, 3 + s, g, 0, False).wait_send()

    return pl.pallas_call(
        body,
        out_shape=jax.ShapeDtypeStruct((N_Z * m_per, n), x.dtype),
        in_specs=[pl.BlockSpec(memory_space=pl.ANY)],
        out_specs=pl.BlockSpec(memory_space=pl.ANY),
        scratch_shapes=[
            pltpu.SemaphoreType.DMA,
            pltpu.SemaphoreType.DMA((N_Z - 1, Q)),
            pltpu.SemaphoreType.DMA((N_Z - 1, Q)),
            pltpu.SemaphoreType.DMA((N_Z - 1, Q)),
            pltpu.SemaphoreType.DMA((N_Z - 1, Q)),
            pltpu.SemaphoreType.DMA((2 * (N_Z - 1), Q)),
            pltpu.SemaphoreType.DMA((2 * (N_Z - 1), Q)),
            pltpu.SemaphoreType.DMA((2 * (N_Z - 1), Q)),
            pltpu.SemaphoreType.DMA((2 * (N_Z - 1), Q)),
            pltpu.SemaphoreType.DMA((2 * (N_Z - 1), QH)),
            pltpu.SemaphoreType.DMA((2 * (N_Z - 1), QH)),
            pltpu.SemaphoreType.DMA((2 * (N_Z - 1), QH)),
            pltpu.SemaphoreType.DMA((2 * (N_Z - 1), QH)),
        ],
        compiler_params=pltpu.CompilerParams(collective_id=0),
    )(x)
